# baseline (speedup 1.0000x reference)
import functools
import math

import jax
import jax.numpy as jnp
from jax import lax
from jax.experimental import pallas as pl
from jax.experimental.pallas import tpu as pltpu
from jax.experimental.pallas import tpu_sc as plsc

_EMB = 32
_SCALE = math.sqrt(_EMB)

_NC = 2
_NS = 16
_NW = _NC * _NS

_B0 = 4096
_B1 = 200
_NT = (_B0 // 128) * (_B1 // 8)
_TPW = _NT // _NW
_CHUNK = 256
_CPW = _TPW * 4


def _emb_kernel(tok_hbm, table_hbm, out_hbm,
                idxraw_v, idx2_a, idx2_b, cb_a, cb_b,
                rows_a, rows_b, stage_v, gsem_a, gsem_b, wsem):
    wid = lax.axis_index("s") * _NC + lax.axis_index("c")
    t0 = wid * _TPW
    iota = lax.iota(jnp.int32, 16)

    idx2 = (idx2_a, idx2_b)
    cb = (cb_a, cb_b)
    rows = (rows_a, rows_b)
    gsem = (gsem_a, gsem_b)

    def start_gather(qi, buf):
        tile = lax.shift_right_logical(qi, 2)
        hh = lax.bitwise_and(qi, 3)

        @pl.when(hh == 0)
        def _():
            pltpu.sync_copy(tok_hbm.at[t0 + tile], idxraw_v)

        @plsc.parallel_loop(0, 16)
        def _prep(i):
            bi = lax.shift_right_logical(i, 3)
            dg = lax.bitwise_and(i, 7)
            r = idxraw_v[hh * 2 + bi, pl.ds(dg * 16, 16)]
            k = i * 16
            idx2[buf][pl.ds(k, 16)] = lax.shift_right_logical(r, 2)
            cb[buf][pl.ds(k, 16)] = lax.bitwise_and(r, 3) * 32

        pltpu.async_copy(table_hbm.at[idx2[buf]], rows[buf], gsem[buf])

    def wait_gather(buf):
        pltpu.make_async_copy(
            table_hbm.at[idx2[buf]], rows[buf], gsem[buf]).wait()

    def drain_writes():
        for i in range(8):
            pltpu.make_async_copy(
                stage_v.at[0, 0], out_hbm.at[0, 0, 0], wsem).wait()

    def do_chunk(qi, buf):
        tile = lax.shift_right_logical(qi, 2)
        hh = lax.bitwise_and(qi, 3)
        q = t0 + tile
        a = lax.shift_right_logical(q, 5)
        c = lax.bitwise_and(q, 31)

        @pl.when(qi + 1 < _CPW)
        def _():
            start_gather(qi + 1, buf ^ 1)

        wait_gather(buf)

        @pl.when(qi > 0)
        def _():
            drain_writes()

        rbuf = rows[buf]
        cbuf = cb[buf]

        @plsc.parallel_loop(0, 16)
        def _shuffle(i):
            bi = lax.shift_right_logical(i, 3)
            dg = lax.bitwise_and(i, 7)
            t = dg * 16
            k = bi * 128 + t
            row = iota + k
            cvec = cbuf[pl.ds(k, 16)]
            for jt in range(4):
                for jr in range(8):
                    col = cvec + (jt * 8 + jr)
                    v = plsc.load_gather(rbuf, [row, col])
                    stage_v[bi, jt, jr, pl.ds(t, 16)] = v * _SCALE

        for bi in range(2):
            for jt in range(4):
                pltpu.make_async_copy(
                    stage_v.at[bi, jt],
                    out_hbm.at[a * 8 + hh * 2 + bi, jt, c],
                    wsem,
                ).start()

    start_gather(0, 0)

    def pair_body(g, carry):
        do_chunk(g * 2, 0)
        do_chunk(g * 2 + 1, 1)
        return carry

    lax.fori_loop(0, _CPW // 2, pair_body, 0)
    drain_writes()


@jax.jit
def _lookup(tok_view, table_view):
    mesh = plsc.VectorSubcoreMesh(core_axis_name="c", subcore_axis_name="s")
    run = functools.partial(
        pl.kernel,
        mesh=mesh,
        out_type=jax.ShapeDtypeStruct((_B1, 4, 32, 8, 128), jnp.float32),
        scratch_types=[
            pltpu.VMEM((8, 128), jnp.int32),
            pltpu.VMEM((_CHUNK,), jnp.int32),
            pltpu.VMEM((_CHUNK,), jnp.int32),
            pltpu.VMEM((_CHUNK,), jnp.int32),
            pltpu.VMEM((_CHUNK,), jnp.int32),
            pltpu.VMEM((_CHUNK, 128), jnp.float32),
            pltpu.VMEM((_CHUNK, 128), jnp.float32),
            pltpu.VMEM((2, 4, 8, 128), jnp.float32),
            pltpu.SemaphoreType.DMA,
            pltpu.SemaphoreType.DMA,
            pltpu.SemaphoreType.DMA,
        ],
        compiler_params=pltpu.CompilerParams(
            use_tc_tiling_on_sc=True, needs_layout_passes=False),
    )(_emb_kernel)
    return run(tok_view, table_view)


def kernel(tokens, table):
    tok_view = (
        tokens.astype(jnp.int32)
        .T.reshape(25, 8, 32, 128)
        .transpose(0, 2, 1, 3)
        .reshape(_NT, 8, 128)
    )
    tab_view = table.reshape(250000, 128)
    out5 = _lookup(tok_view, tab_view)
    return (
        out5.transpose(2, 4, 0, 1, 3)
        .reshape(_B0, _B1, _EMB)
    )


# speedup vs baseline: 1.5309x; 1.5309x over previous
import functools
import math

import jax
import jax.numpy as jnp
from jax import lax
from jax.experimental import pallas as pl
from jax.experimental.pallas import tpu as pltpu
from jax.experimental.pallas import tpu_sc as plsc

_EMB = 32
_SCALE = math.sqrt(_EMB)

_NC = 2
_NS = 16
_NW = _NC * _NS

_B0 = 4096
_B1 = 200
_CHUNK = 1024
_NQ = (_B0 // 128) * (_B1 // 8)
_CPW = _NQ // _NW
_PITCH = 129


def _emb_kernel(tok_hbm, table_hbm, out_hbm, idx_v, rows_v, stage_v,
                gsem, wsem):
    wid = lax.axis_index("s") * _NC + lax.axis_index("c")
    q0 = wid * _CPW
    iota = lax.iota(jnp.int32, 16)
    jv0 = iota
    jv1 = iota + 16

    def start_gather(qi, buf):
        pltpu.sync_copy(tok_hbm.at[q0 + qi], idx_v.at[buf])
        pltpu.async_copy(table_hbm.at[idx_v.at[buf]], rows_v.at[buf],
                         gsem.at[buf])

    def wait_gather(buf):
        pltpu.make_async_copy(
            table_hbm.at[idx_v.at[buf]], rows_v.at[buf], gsem.at[buf]).wait()

    def drain_writes():
        for b in range(8):
            for jt in range(4):
                pltpu.make_async_copy(
                    stage_v.at[0, pl.ds(0, 8), pl.ds(0, 128)],
                    out_hbm.at[0, 0, 0],
                    wsem,
                ).wait()

    def do_chunk(qi, buf):
        q = q0 + qi
        a = lax.shift_right_logical(q, 5)
        c = lax.bitwise_and(q, 31)

        @pl.when(qi + 1 < _CPW)
        def _():
            start_gather(qi + 1, buf ^ 1)

        wait_gather(buf)

        @pl.when(qi > 0)
        def _():
            drain_writes()

        rows = rows_v.at[buf]

        @plsc.parallel_loop(0, 256)
        def _shuffle(i):
            for u in range(4):
                k = i * 4 + u
                b = lax.shift_right_logical(k, 7)
                d = lax.bitwise_and(k, 127)
                bv = jnp.full((16,), b, jnp.int32)
                dv = jnp.full((16,), d, jnp.int32)
                v0 = rows[k, pl.ds(0, 16)] * _SCALE
                v1 = rows[k, pl.ds(16, 16)] * _SCALE
                plsc.store_scatter(stage_v, [bv, jv0, dv], v0)
                plsc.store_scatter(stage_v, [bv, jv1, dv], v1)

        for b in range(8):
            for jt in range(4):
                pltpu.make_async_copy(
                    stage_v.at[b, pl.ds(jt * 8, 8), pl.ds(0, 128)],
                    out_hbm.at[a * 8 + b, jt, c],
                    wsem,
                ).start()

    start_gather(0, 0)

    def pair_body(g, carry):
        do_chunk(g * 2, 0)
        do_chunk(g * 2 + 1, 1)
        return carry

    lax.fori_loop(0, _CPW // 2, pair_body, 0)
    do_chunk(_CPW - 1, 0)
    drain_writes()


@jax.jit
def _lookup(tok_view, table):
    mesh = plsc.VectorSubcoreMesh(core_axis_name="c", subcore_axis_name="s")
    run = functools.partial(
        pl.kernel,
        mesh=mesh,
        out_type=jax.ShapeDtypeStruct((_B1, 4, 32, 8, 128), jnp.float32),
        scratch_types=[
            pltpu.VMEM((2, _CHUNK), jnp.int32),
            pltpu.VMEM((2, _CHUNK, _EMB), jnp.float32),
            pltpu.VMEM((8, 32, _PITCH), jnp.float32),
            pltpu.SemaphoreType.DMA((2,)),
            pltpu.SemaphoreType.DMA,
        ],
        compiler_params=pltpu.CompilerParams(
            use_tc_tiling_on_sc=False, needs_layout_passes=False),
    )(_emb_kernel)
    return run(tok_view, table)


def kernel(tokens, table):
    tok_view = (
        tokens.astype(jnp.int32)
        .T.reshape(25, 8, 32, 128)
        .transpose(0, 2, 1, 3)
        .reshape(_NQ, _CHUNK)
    )
    out5 = _lookup(tok_view, table)
    return (
        out5.transpose(2, 4, 0, 1, 3)
        .reshape(_B0, _B1, _EMB)
    )
